# baseline (speedup 1.0000x reference)
import functools

import jax
import jax.numpy as jnp
from jax import lax
from jax.experimental import pallas as pl
from jax.experimental.pallas import tpu as pltpu
from jax.experimental.pallas import tpu_sc as plsc

VOCAB = 30000
D_EMB = 512
D_OUT = 64
EPS = 1e-5

ROW_BLOCK = 1000


def _table_body(emb_ref, gamma_ref, beta_ref, wt_ref, b_ref, out_ref):
    e = emb_ref[...]
    mean = jnp.mean(e, axis=1, keepdims=True)
    c = e - mean
    var = jnp.mean(c * c, axis=1, keepdims=True)
    h = c * lax.rsqrt(var + EPS) * gamma_ref[...] + beta_ref[...]
    z = jnp.dot(h, wt_ref[...], preferred_element_type=jnp.float32) + b_ref[...]
    z = z - jnp.max(z, axis=1, keepdims=True)
    ez = jnp.exp(z)
    out_ref[...] = ez / jnp.sum(ez, axis=1, keepdims=True)


def _make_table(emb, gamma, beta, W, b):
    nblk = VOCAB // ROW_BLOCK
    return pl.pallas_call(
        _table_body,
        grid=(nblk,),
        in_specs=[
            pl.BlockSpec((ROW_BLOCK, D_EMB), lambda i: (i, 0)),
            pl.BlockSpec((1, D_EMB), lambda i: (0, 0)),
            pl.BlockSpec((1, D_EMB), lambda i: (0, 0)),
            pl.BlockSpec((D_EMB, D_OUT), lambda i: (0, 0)),
            pl.BlockSpec((1, D_OUT), lambda i: (0, 0)),
        ],
        out_specs=pl.BlockSpec((ROW_BLOCK, D_OUT), lambda i: (i, 0)),
        out_shape=jax.ShapeDtypeStruct((VOCAB, D_OUT), jnp.float32),
    )(emb, gamma.reshape(1, D_EMB), beta.reshape(1, D_EMB),
      W.T, b.reshape(1, D_OUT))


try:
    _INFO = plsc.get_sparse_core_info()
    _NC, _NS = _INFO.num_cores, _INFO.num_subcores
except ValueError:
    _NC, _NS = 2, 16
_NW = _NC * _NS
_CA = 104
_CB = 96


def _gather_body(rows_per_w, L, table_hbm, x_hbm, out_hbm,
                 idx_v, rows_a0, rows_b0, rows_a1, rows_b1,
                 sem_a0, sem_b0, sem_a1, sem_b1):
    wid = lax.axis_index("s") * _NC + lax.axis_index("c")
    r0 = wid * rows_per_w
    pltpu.sync_copy(x_hbm.at[pl.ds(r0, rows_per_w)], idx_v)

    def issue(r, ra, rb, sa, sb):
        pltpu.async_copy(table_hbm.at[idx_v.at[r, pl.ds(0, _CA)]], ra, sa)
        pltpu.async_copy(table_hbm.at[idx_v.at[r, pl.ds(_CA, _CB)]], rb, sb)

    def drain(r, ra, rb, sa, sb):
        row = r0 + r
        pltpu.make_async_copy(table_hbm.at[idx_v.at[r, pl.ds(0, _CA)]],
                              ra, sa).wait()
        pltpu.sync_copy(ra, out_hbm.at[row, pl.ds(0, _CA)])
        pltpu.make_async_copy(table_hbm.at[idx_v.at[r, pl.ds(_CA, _CB)]],
                              rb, sb).wait()
        pltpu.sync_copy(rb, out_hbm.at[row, pl.ds(_CA, _CB)])

    issue(0, rows_a0, rows_b0, sem_a0, sem_b0)

    def step(j, _):
        issue(2 * j + 1, rows_a1, rows_b1, sem_a1, sem_b1)
        drain(2 * j, rows_a0, rows_b0, sem_a0, sem_b0)
        issue(2 * j + 2, rows_a0, rows_b0, sem_a0, sem_b0)
        drain(2 * j + 1, rows_a1, rows_b1, sem_a1, sem_b1)
        return 0

    lax.fori_loop(0, rows_per_w // 2 - 1, step, 0, unroll=False)
    last = rows_per_w - 2
    issue(last + 1, rows_a1, rows_b1, sem_a1, sem_b1)
    drain(last, rows_a0, rows_b0, sem_a0, sem_b0)
    drain(last + 1, rows_a1, rows_b1, sem_a1, sem_b1)


def _gather(table, x):
    B, L = x.shape
    rows_per_w = B // _NW
    mesh = plsc.VectorSubcoreMesh(core_axis_name="c", subcore_axis_name="s")
    k = pl.kernel(
        functools.partial(_gather_body, rows_per_w, L),
        out_type=jax.ShapeDtypeStruct((B, L, D_OUT), jnp.float32),
        mesh=mesh,
        scratch_types=[
            pltpu.VMEM((rows_per_w, L), jnp.int32),
            pltpu.VMEM((_CA, D_OUT), jnp.float32),
            pltpu.VMEM((_CB, D_OUT), jnp.float32),
            pltpu.VMEM((_CA, D_OUT), jnp.float32),
            pltpu.VMEM((_CB, D_OUT), jnp.float32),
            pltpu.SemaphoreType.DMA,
            pltpu.SemaphoreType.DMA,
            pltpu.SemaphoreType.DMA,
            pltpu.SemaphoreType.DMA,
        ],
        compiler_params=pltpu.CompilerParams(use_tc_tiling_on_sc=False),
    )
    return k(table, x)


def kernel(x, emb, gamma, beta, W, b):
    table = _make_table(emb, gamma, beta, W, b)
    return _gather(table, x.astype(jnp.int32))


# speedup vs baseline: 1.0430x; 1.0430x over previous
import functools

import jax
import jax.numpy as jnp
from jax import lax
from jax.experimental import pallas as pl
from jax.experimental.pallas import tpu as pltpu
from jax.experimental.pallas import tpu_sc as plsc

VOCAB = 30000
D_EMB = 512
D_OUT = 64
EPS = 1e-5

ROW_BLOCK = 3000


def _table_body(emb_ref, gamma_ref, beta_ref, wt_ref, b_ref, out_ref):
    e = emb_ref[...]
    mean = jnp.mean(e, axis=1, keepdims=True)
    c = e - mean
    var = jnp.mean(c * c, axis=1, keepdims=True)
    h = c * lax.rsqrt(var + EPS) * gamma_ref[...] + beta_ref[...]
    z = jnp.dot(h, wt_ref[...], preferred_element_type=jnp.float32) + b_ref[...]
    z = z - jnp.max(z, axis=1, keepdims=True)
    ez = jnp.exp(z)
    out_ref[...] = ez / jnp.sum(ez, axis=1, keepdims=True)


def _make_table(emb, gamma, beta, W, b):
    nblk = VOCAB // ROW_BLOCK
    return pl.pallas_call(
        _table_body,
        grid=(nblk,),
        in_specs=[
            pl.BlockSpec((ROW_BLOCK, D_EMB), lambda i: (i, 0)),
            pl.BlockSpec((1, D_EMB), lambda i: (0, 0)),
            pl.BlockSpec((1, D_EMB), lambda i: (0, 0)),
            pl.BlockSpec((D_EMB, D_OUT), lambda i: (0, 0)),
            pl.BlockSpec((1, D_OUT), lambda i: (0, 0)),
        ],
        out_specs=pl.BlockSpec((ROW_BLOCK, D_OUT), lambda i: (i, 0)),
        out_shape=jax.ShapeDtypeStruct((VOCAB, D_OUT), jnp.float32),
    )(emb, gamma.reshape(1, D_EMB), beta.reshape(1, D_EMB),
      W.T, b.reshape(1, D_OUT))


try:
    _INFO = plsc.get_sparse_core_info()
    _NC, _NS = _INFO.num_cores, _INFO.num_subcores
except ValueError:
    _NC, _NS = 2, 16
_NW = _NC * _NS
_CA = 104
_CB = 96


def _gather_body(rows_per_w, L, table_hbm, x_hbm, out_hbm,
                 idx_v, rows_a0, rows_b0, rows_a1, rows_b1,
                 sem_a0, sem_b0, sem_a1, sem_b1):
    wid = lax.axis_index("s") * _NC + lax.axis_index("c")
    r0 = wid * rows_per_w
    pltpu.sync_copy(x_hbm.at[pl.ds(r0, rows_per_w)], idx_v)

    def issue(r, ra, rb, sa, sb):
        pltpu.async_copy(table_hbm.at[idx_v.at[r, pl.ds(0, _CA)]], ra, sa)
        pltpu.async_copy(table_hbm.at[idx_v.at[r, pl.ds(_CA, _CB)]], rb, sb)

    def drain(r, ra, rb, sa, sb):
        row = r0 + r
        pltpu.make_async_copy(table_hbm.at[idx_v.at[r, pl.ds(0, _CA)]],
                              ra, sa).wait()
        pltpu.sync_copy(ra, out_hbm.at[row, pl.ds(0, _CA)])
        pltpu.make_async_copy(table_hbm.at[idx_v.at[r, pl.ds(_CA, _CB)]],
                              rb, sb).wait()
        pltpu.sync_copy(rb, out_hbm.at[row, pl.ds(_CA, _CB)])

    issue(0, rows_a0, rows_b0, sem_a0, sem_b0)

    def step(j, _):
        issue(2 * j + 1, rows_a1, rows_b1, sem_a1, sem_b1)
        drain(2 * j, rows_a0, rows_b0, sem_a0, sem_b0)
        issue(2 * j + 2, rows_a0, rows_b0, sem_a0, sem_b0)
        drain(2 * j + 1, rows_a1, rows_b1, sem_a1, sem_b1)
        return 0

    lax.fori_loop(0, rows_per_w // 2 - 1, step, 0, unroll=False)
    last = rows_per_w - 2
    issue(last + 1, rows_a1, rows_b1, sem_a1, sem_b1)
    drain(last, rows_a0, rows_b0, sem_a0, sem_b0)
    drain(last + 1, rows_a1, rows_b1, sem_a1, sem_b1)


def _gather(table, x):
    B, L = x.shape
    rows_per_w = B // _NW
    mesh = plsc.VectorSubcoreMesh(core_axis_name="c", subcore_axis_name="s")
    k = pl.kernel(
        functools.partial(_gather_body, rows_per_w, L),
        out_type=jax.ShapeDtypeStruct((B, L, D_OUT), jnp.float32),
        mesh=mesh,
        scratch_types=[
            pltpu.VMEM((rows_per_w, L), jnp.int32),
            pltpu.VMEM((_CA, D_OUT), jnp.float32),
            pltpu.VMEM((_CB, D_OUT), jnp.float32),
            pltpu.VMEM((_CA, D_OUT), jnp.float32),
            pltpu.VMEM((_CB, D_OUT), jnp.float32),
            pltpu.SemaphoreType.DMA,
            pltpu.SemaphoreType.DMA,
            pltpu.SemaphoreType.DMA,
            pltpu.SemaphoreType.DMA,
        ],
        compiler_params=pltpu.CompilerParams(use_tc_tiling_on_sc=False),
    )
    return k(table, x)


def kernel(x, emb, gamma, beta, W, b):
    table = _make_table(emb, gamma, beta, W, b)
    return _gather(table, x.astype(jnp.int32))


# speedup vs baseline: 1.0472x; 1.0041x over previous
import functools

import jax
import jax.numpy as jnp
from jax import lax
from jax.experimental import pallas as pl
from jax.experimental.pallas import tpu as pltpu
from jax.experimental.pallas import tpu_sc as plsc

VOCAB = 30000
D_EMB = 512
D_OUT = 64
EPS = 1e-5

ROW_BLOCK = 6000


def _table_body(emb_ref, gamma_ref, beta_ref, wt_ref, b_ref, out_ref):
    e = emb_ref[...]
    mean = jnp.mean(e, axis=1, keepdims=True)
    c = e - mean
    var = jnp.mean(c * c, axis=1, keepdims=True)
    h = c * lax.rsqrt(var + EPS) * gamma_ref[...] + beta_ref[...]
    z = jnp.dot(h, wt_ref[...], preferred_element_type=jnp.float32) + b_ref[...]
    z = z - jnp.max(z, axis=1, keepdims=True)
    ez = jnp.exp(z)
    out_ref[...] = ez / jnp.sum(ez, axis=1, keepdims=True)


def _make_table(emb, gamma, beta, W, b):
    nblk = VOCAB // ROW_BLOCK
    return pl.pallas_call(
        _table_body,
        grid=(nblk,),
        in_specs=[
            pl.BlockSpec((ROW_BLOCK, D_EMB), lambda i: (i, 0)),
            pl.BlockSpec((1, D_EMB), lambda i: (0, 0)),
            pl.BlockSpec((1, D_EMB), lambda i: (0, 0)),
            pl.BlockSpec((D_EMB, D_OUT), lambda i: (0, 0)),
            pl.BlockSpec((1, D_OUT), lambda i: (0, 0)),
        ],
        out_specs=pl.BlockSpec((ROW_BLOCK, D_OUT), lambda i: (i, 0)),
        out_shape=jax.ShapeDtypeStruct((VOCAB, D_OUT), jnp.float32),
    )(emb, gamma.reshape(1, D_EMB), beta.reshape(1, D_EMB),
      W.T, b.reshape(1, D_OUT))


try:
    _INFO = plsc.get_sparse_core_info()
    _NC, _NS = _INFO.num_cores, _INFO.num_subcores
except ValueError:
    _NC, _NS = 2, 16
_NW = _NC * _NS
_CA = 104
_CB = 96


def _gather_body(rows_per_w, L, table_hbm, x_hbm, out_hbm,
                 idx_v, rows_a0, rows_b0, rows_a1, rows_b1,
                 sem_a0, sem_b0, sem_a1, sem_b1):
    wid = lax.axis_index("s") * _NC + lax.axis_index("c")
    r0 = wid * rows_per_w
    pltpu.sync_copy(x_hbm.at[pl.ds(r0, rows_per_w)], idx_v)

    def issue(r, ra, rb, sa, sb):
        pltpu.async_copy(table_hbm.at[idx_v.at[r, pl.ds(0, _CA)]], ra, sa)
        pltpu.async_copy(table_hbm.at[idx_v.at[r, pl.ds(_CA, _CB)]], rb, sb)

    def drain(r, ra, rb, sa, sb):
        row = r0 + r
        pltpu.make_async_copy(table_hbm.at[idx_v.at[r, pl.ds(0, _CA)]],
                              ra, sa).wait()
        pltpu.sync_copy(ra, out_hbm.at[row, pl.ds(0, _CA)])
        pltpu.make_async_copy(table_hbm.at[idx_v.at[r, pl.ds(_CA, _CB)]],
                              rb, sb).wait()
        pltpu.sync_copy(rb, out_hbm.at[row, pl.ds(_CA, _CB)])

    issue(0, rows_a0, rows_b0, sem_a0, sem_b0)

    def step(j, _):
        issue(2 * j + 1, rows_a1, rows_b1, sem_a1, sem_b1)
        drain(2 * j, rows_a0, rows_b0, sem_a0, sem_b0)
        issue(2 * j + 2, rows_a0, rows_b0, sem_a0, sem_b0)
        drain(2 * j + 1, rows_a1, rows_b1, sem_a1, sem_b1)
        return 0

    lax.fori_loop(0, rows_per_w // 2 - 1, step, 0, unroll=False)
    last = rows_per_w - 2
    issue(last + 1, rows_a1, rows_b1, sem_a1, sem_b1)
    drain(last, rows_a0, rows_b0, sem_a0, sem_b0)
    drain(last + 1, rows_a1, rows_b1, sem_a1, sem_b1)


def _gather(table, x):
    B, L = x.shape
    rows_per_w = B // _NW
    mesh = plsc.VectorSubcoreMesh(core_axis_name="c", subcore_axis_name="s")
    k = pl.kernel(
        functools.partial(_gather_body, rows_per_w, L),
        out_type=jax.ShapeDtypeStruct((B, L, D_OUT), jnp.float32),
        mesh=mesh,
        scratch_types=[
            pltpu.VMEM((rows_per_w, L), jnp.int32),
            pltpu.VMEM((_CA, D_OUT), jnp.float32),
            pltpu.VMEM((_CB, D_OUT), jnp.float32),
            pltpu.VMEM((_CA, D_OUT), jnp.float32),
            pltpu.VMEM((_CB, D_OUT), jnp.float32),
            pltpu.SemaphoreType.DMA,
            pltpu.SemaphoreType.DMA,
            pltpu.SemaphoreType.DMA,
            pltpu.SemaphoreType.DMA,
        ],
        compiler_params=pltpu.CompilerParams(use_tc_tiling_on_sc=False),
    )
    return k(table, x)


def kernel(x, emb, gamma, beta, W, b):
    table = _make_table(emb, gamma, beta, W, b)
    return _gather(table, x.astype(jnp.int32))
